# Initial kernel scaffold; baseline (speedup 1.0000x reference)
import jax
import jax.numpy as jnp
from jax.experimental import pallas as pl


def kernel(x, edge_index, batch, W1, b1, W2, b2, W3, b3, g1, bt1, g2, bt2, g3, bt3, fc1W, fc1b, fc2W, fc2b):
    raise NotImplementedError("write your pallas kernel here")


# speedup vs baseline: 6.5298x; 6.5298x over previous
import functools

import jax
import jax.numpy as jnp
from jax import lax
from jax.experimental import pallas as pl
from jax.experimental.pallas import tpu as pltpu
from jax.experimental.pallas import tpu_sc as plsc

N = 10000
E = 320000
D = 128
G = 16
NC = 2
NS = 16
NW = NC * NS
EPW = E // NW
CH = 80
NCHUNK = EPW // CH
RPS = 624
TAIL = N - NS * RPS

_mesh = plsc.VectorSubcoreMesh(core_axis_name="c", subcore_axis_name="s")


def _sc_agg_body(h_hbm, src_hbm, dst_hbm, zero_hbm, out_hbm,
                 src_v, dst_v, rows_v, acc_sh, sem):
    c = lax.axis_index("c")
    s = lax.axis_index("s")
    w = c * NS + s
    pltpu.sync_copy(zero_hbm.at[pl.ds(s * RPS, RPS)],
                    acc_sh.at[pl.ds(s * RPS, RPS)])

    @pl.when(s == NS - 1)
    def _():
        pltpu.sync_copy(zero_hbm.at[pl.ds(NS * RPS, TAIL)],
                        acc_sh.at[pl.ds(NS * RPS, TAIL)])
    pltpu.sync_copy(src_hbm.at[w], src_v)
    pltpu.sync_copy(dst_hbm.at[w], dst_v)
    plsc.subcore_barrier()

    @pl.loop(0, NCHUNK)
    def _(j):
        pltpu.async_copy(h_hbm.at[src_v.at[j]], rows_v, sem).wait()
        pltpu.sync_copy(rows_v, acc_sh.at[dst_v.at[j]], add=True)

    plsc.subcore_barrier()
    pltpu.sync_copy(acc_sh.at[pl.ds(s * RPS, RPS)],
                    out_hbm.at[c, pl.ds(s * RPS, RPS)])

    @pl.when(s == NS - 1)
    def _():
        pltpu.sync_copy(acc_sh.at[pl.ds(NS * RPS, TAIL)],
                        out_hbm.at[c, pl.ds(NS * RPS, TAIL)])


def _sc_agg(h, src3, dst3, zeros):
    k = pl.kernel(
        _sc_agg_body,
        out_type=jax.ShapeDtypeStruct((NC, N, D), jnp.float32),
        mesh=_mesh,
        scratch_types=[
            pltpu.VMEM((NCHUNK, CH), jnp.int32),
            pltpu.VMEM((NCHUNK, CH), jnp.int32),
            pltpu.VMEM((CH, D), jnp.float32),
            pltpu.VMEM_SHARED((N, D), jnp.float32),
            pltpu.SemaphoreType.DMA,
        ],
        name="sc_gin_agg",
    )
    return k(h, src3, dst3, zeros)


def _bn_relu(z, g, bt):
    m = jnp.mean(z, axis=0, keepdims=True)
    zc = z - m
    v = jnp.mean(zc * zc, axis=0, keepdims=True)
    y = g * zc * lax.rsqrt(v + 1e-5) + bt
    return jnp.maximum(y, 0.0)


def _tc_layer_body(residual, h_ref, p_ref, w_ref, b_ref, g_ref, bt_ref, o_ref):
    h = h_ref[...]
    z = h + p_ref[0] + p_ref[1]
    z = jnp.dot(z, w_ref[...], preferred_element_type=jnp.float32,
                precision=lax.Precision.HIGHEST) + b_ref[...]
    y = _bn_relu(z, g_ref[...], bt_ref[...])
    o_ref[...] = y + h if residual else y


def _tc_layer(h, p, w, b, g, bt, residual):
    return pl.pallas_call(
        functools.partial(_tc_layer_body, residual),
        out_shape=jax.ShapeDtypeStruct((N, D), jnp.float32),
    )(h, p, w, b.reshape(1, D), g.reshape(1, D), bt.reshape(1, D))


def _tc_final_body(h_ref, p_ref, w_ref, b_ref, g_ref, bt_ref, batch_ref,
                   fc1w_ref, fc1b_ref, fc2w_ref, fc2b_ref, o_ref):
    h = h_ref[...]
    z = h + p_ref[0] + p_ref[1]
    z = jnp.dot(z, w_ref[...], preferred_element_type=jnp.float32,
                precision=lax.Precision.HIGHEST) + b_ref[...]
    x3 = _bn_relu(z, g_ref[...], bt_ref[...]) + h
    seg = lax.broadcasted_iota(jnp.int32, (1, G), 1)
    mask = (batch_ref[...] == seg).astype(jnp.float32)
    cnt = jnp.sum(mask, axis=0, keepdims=True)
    mask_n = mask / jnp.maximum(cnt, 1.0)
    pooled = lax.dot_general(mask_n, x3, (((0,), (0,)), ((), ())),
                             preferred_element_type=jnp.float32,
                             precision=lax.Precision.HIGHEST)
    hfc = jnp.maximum(
        jnp.dot(pooled, fc1w_ref[...], preferred_element_type=jnp.float32,
                precision=lax.Precision.HIGHEST) + fc1b_ref[...], 0.0)
    o_ref[...] = jnp.dot(hfc, fc2w_ref[...],
                         preferred_element_type=jnp.float32,
                         precision=lax.Precision.HIGHEST) + fc2b_ref[...]


def _tc_final(h, p, w, b, g, bt, batch, fc1w, fc1b, fc2w, fc2b):
    return pl.pallas_call(
        _tc_final_body,
        out_shape=jax.ShapeDtypeStruct((G, 1), jnp.float32),
    )(h, p, w, b.reshape(1, D), g.reshape(1, D), bt.reshape(1, D),
      batch.reshape(N, 1), fc1w, fc1b.reshape(1, D), fc2w, fc2b.reshape(1, 1))


def kernel(x, edge_index, batch, W1, b1, W2, b2, W3, b3,
           g1, bt1, g2, bt2, g3, bt3, fc1W, fc1b, fc2W, fc2b):
    src3 = edge_index[0].reshape(NW, NCHUNK, CH)
    dst3 = edge_index[1].reshape(NW, NCHUNK, CH)
    zeros = jnp.zeros((N, D), jnp.float32)

    p1 = _sc_agg(x, src3, dst3, zeros)
    x1 = _tc_layer(x, p1, W1, b1, g1, bt1, residual=False)
    p2 = _sc_agg(x1, src3, dst3, zeros)
    x2 = _tc_layer(x1, p2, W2, b2, g2, bt2, residual=True)
    p3 = _sc_agg(x2, src3, dst3, zeros)
    return _tc_final(x2, p3, W3, b3, g3, bt3, batch,
                     fc1W, fc1b, fc2W, fc2b)


# speedup vs baseline: 9.7891x; 1.4991x over previous
import functools

import jax
import jax.numpy as jnp
from jax import lax
from jax.experimental import pallas as pl
from jax.experimental.pallas import tpu as pltpu
from jax.experimental.pallas import tpu_sc as plsc

N = 10000
E = 320000
D = 128
G = 16
NC = 2
NS = 16
NW = NC * NS
EPW = E // NW
CH = 80
NCHUNK = EPW // CH
NBLK = 5
CPB = NCHUNK // NBLK
RPS = 624
TAIL = N - NS * RPS

_mesh = plsc.VectorSubcoreMesh(core_axis_name="c", subcore_axis_name="s")


def _sc_agg_body(h_hbm, src_hbm, dst_hbm, zero_hbm, out_hbm,
                 src_v, dst_v, rows_a, rows_b, acc_sh, sem_a, sem_b):
    c = lax.axis_index("c")
    s = lax.axis_index("s")
    w = c * NS + s
    pltpu.sync_copy(zero_hbm.at[pl.ds(s * RPS, RPS)],
                    acc_sh.at[pl.ds(s * RPS, RPS)])

    @pl.when(s == NS - 1)
    def _():
        pltpu.sync_copy(zero_hbm.at[pl.ds(NS * RPS, TAIL)],
                        acc_sh.at[pl.ds(NS * RPS, TAIL)])
    plsc.subcore_barrier()

    def gather(j, buf, sem):
        pltpu.async_copy(h_hbm.at[src_v.at[j]], buf, sem)

    def gwait(buf, sem):
        pltpu.make_async_copy(zero_hbm.at[pl.ds(0, CH)], buf, sem).wait()

    def scat(j, buf):
        pltpu.sync_copy(buf, acc_sh.at[dst_v.at[j]], add=True)

    @pl.loop(0, NBLK)
    def _(blk):
        pltpu.sync_copy(src_hbm.at[w, blk], src_v)
        pltpu.sync_copy(dst_hbm.at[w, blk], dst_v)
        gather(0, rows_a, sem_a)

        @pl.loop(0, (CPB - 1) // 2)
        def _(k):
            j = 2 * k
            gather(j + 1, rows_b, sem_b)
            gwait(rows_a, sem_a)
            scat(j, rows_a)
            gather(j + 2, rows_a, sem_a)
            gwait(rows_b, sem_b)
            scat(j + 1, rows_b)

        gwait(rows_a, sem_a)
        scat(CPB - 1, rows_a)

    plsc.subcore_barrier()
    pltpu.sync_copy(acc_sh.at[pl.ds(s * RPS, RPS)],
                    out_hbm.at[c, pl.ds(s * RPS, RPS)])

    @pl.when(s == NS - 1)
    def _():
        pltpu.sync_copy(acc_sh.at[pl.ds(NS * RPS, TAIL)],
                        out_hbm.at[c, pl.ds(NS * RPS, TAIL)])


def _sc_agg(h, src3, dst3, zeros):
    k = pl.kernel(
        _sc_agg_body,
        out_type=jax.ShapeDtypeStruct((NC, N, D), jnp.float32),
        mesh=_mesh,
        scratch_types=[
            pltpu.VMEM((CPB, CH), jnp.int32),
            pltpu.VMEM((CPB, CH), jnp.int32),
            pltpu.VMEM((CH, D), jnp.float32),
            pltpu.VMEM((CH, D), jnp.float32),
            pltpu.VMEM_SHARED((N, D), jnp.float32),
            pltpu.SemaphoreType.DMA,
            pltpu.SemaphoreType.DMA,
        ],
        name="sc_gin_agg",
    )
    return k(h, src3, dst3, zeros)


def _bn_relu(z, g, bt):
    m = jnp.mean(z, axis=0, keepdims=True)
    zc = z - m
    v = jnp.mean(zc * zc, axis=0, keepdims=True)
    y = g * zc * lax.rsqrt(v + 1e-5) + bt
    return jnp.maximum(y, 0.0)


def _tc_layer_body(residual, h_ref, p_ref, w_ref, b_ref, g_ref, bt_ref, o_ref):
    h = h_ref[...]
    z = h + p_ref[0] + p_ref[1]
    z = jnp.dot(z, w_ref[...], preferred_element_type=jnp.float32,
                precision=lax.Precision.HIGHEST) + b_ref[...]
    y = _bn_relu(z, g_ref[...], bt_ref[...])
    o_ref[...] = y + h if residual else y


def _tc_layer(h, p, w, b, g, bt, residual):
    return pl.pallas_call(
        functools.partial(_tc_layer_body, residual),
        out_shape=jax.ShapeDtypeStruct((N, D), jnp.float32),
    )(h, p, w, b.reshape(1, D), g.reshape(1, D), bt.reshape(1, D))


def _tc_final_body(h_ref, p_ref, w_ref, b_ref, g_ref, bt_ref, batch_ref,
                   fc1w_ref, fc1b_ref, fc2w_ref, fc2b_ref, o_ref):
    h = h_ref[...]
    z = h + p_ref[0] + p_ref[1]
    z = jnp.dot(z, w_ref[...], preferred_element_type=jnp.float32,
                precision=lax.Precision.HIGHEST) + b_ref[...]
    x3 = _bn_relu(z, g_ref[...], bt_ref[...]) + h
    seg = lax.broadcasted_iota(jnp.int32, (1, G), 1)
    mask = (batch_ref[...] == seg).astype(jnp.float32)
    cnt = jnp.sum(mask, axis=0, keepdims=True)
    mask_n = mask / jnp.maximum(cnt, 1.0)
    pooled = lax.dot_general(mask_n, x3, (((0,), (0,)), ((), ())),
                             preferred_element_type=jnp.float32,
                             precision=lax.Precision.HIGHEST)
    hfc = jnp.maximum(
        jnp.dot(pooled, fc1w_ref[...], preferred_element_type=jnp.float32,
                precision=lax.Precision.HIGHEST) + fc1b_ref[...], 0.0)
    o_ref[...] = jnp.dot(hfc, fc2w_ref[...],
                         preferred_element_type=jnp.float32,
                         precision=lax.Precision.HIGHEST) + fc2b_ref[...]


def _tc_final(h, p, w, b, g, bt, batch, fc1w, fc1b, fc2w, fc2b):
    return pl.pallas_call(
        _tc_final_body,
        out_shape=jax.ShapeDtypeStruct((G, 1), jnp.float32),
    )(h, p, w, b.reshape(1, D), g.reshape(1, D), bt.reshape(1, D),
      batch.reshape(N, 1), fc1w, fc1b.reshape(1, D), fc2w, fc2b.reshape(1, 1))


def kernel(x, edge_index, batch, W1, b1, W2, b2, W3, b3,
           g1, bt1, g2, bt2, g3, bt3, fc1W, fc1b, fc2W, fc2b):
    src3 = edge_index[0].reshape(NW, NBLK, CPB, CH)
    dst3 = edge_index[1].reshape(NW, NBLK, CPB, CH)
    zeros = jnp.zeros((N, D), jnp.float32)

    p1 = _sc_agg(x, src3, dst3, zeros)
    x1 = _tc_layer(x, p1, W1, b1, g1, bt1, residual=False)
    p2 = _sc_agg(x1, src3, dst3, zeros)
    x2 = _tc_layer(x1, p2, W2, b2, g2, bt2, residual=True)
    p3 = _sc_agg(x2, src3, dst3, zeros)
    return _tc_final(x2, p3, W3, b3, g3, bt3, batch,
                     fc1W, fc1b, fc2W, fc2b)


# speedup vs baseline: 10.4873x; 1.0713x over previous
import functools

import jax
import jax.numpy as jnp
from jax import lax
from jax.experimental import pallas as pl
from jax.experimental.pallas import tpu as pltpu
from jax.experimental.pallas import tpu_sc as plsc

N = 10000
E = 320000
D = 128
G = 16
NC = 2
NS = 16
NW = NC * NS
CH = 128
PAD = 64
EPW = 10240
EPAD = EPW - E // NW
NCHUNK = EPW // CH
NBLK = 4
CPB = NCHUNK // NBLK
RPS = 624
TAIL = N - NS * RPS

_mesh = plsc.VectorSubcoreMesh(core_axis_name="c", subcore_axis_name="s")


def _sc_agg_body(h_hbm, src_hbm, dst_hbm, zero_hbm, out_hbm,
                 src_v, dst_v, rows_a, rows_b, acc_sh, sem_a, sem_b):
    c = lax.axis_index("c")
    s = lax.axis_index("s")
    w = c * NS + s
    pltpu.sync_copy(zero_hbm.at[pl.ds(s * RPS, RPS)],
                    acc_sh.at[pl.ds(s * RPS, RPS)])

    @pl.when(s == NS - 1)
    def _():
        pltpu.sync_copy(zero_hbm.at[pl.ds(NS * RPS, TAIL)],
                        acc_sh.at[pl.ds(NS * RPS, TAIL)])
    plsc.subcore_barrier()

    def gather(j, buf, sem):
        pltpu.async_copy(h_hbm.at[src_v.at[j]], buf, sem)

    def gwait(buf, sem):
        pltpu.make_async_copy(zero_hbm.at[pl.ds(0, CH)], buf, sem).wait()

    def scat(j, buf):
        pltpu.sync_copy(buf, acc_sh.at[dst_v.at[j]], add=True)

    @pl.loop(0, NBLK)
    def _(blk):
        pltpu.sync_copy(src_hbm.at[w, blk], src_v)
        pltpu.sync_copy(dst_hbm.at[w, blk], dst_v)
        gather(0, rows_a, sem_a)
        gather(1, rows_b, sem_b)

        @pl.loop(0, CPB // 2 - 1)
        def _(k):
            j = 2 * k
            gwait(rows_a, sem_a)
            scat(j, rows_a)
            gather(j + 2, rows_a, sem_a)
            gwait(rows_b, sem_b)
            scat(j + 1, rows_b)
            gather(j + 3, rows_b, sem_b)

        gwait(rows_a, sem_a)
        scat(CPB - 2, rows_a)
        gwait(rows_b, sem_b)
        scat(CPB - 1, rows_b)

    plsc.subcore_barrier()
    pltpu.sync_copy(acc_sh.at[pl.ds(s * RPS, RPS)],
                    out_hbm.at[c, pl.ds(s * RPS, RPS)])

    @pl.when(s == NS - 1)
    def _():
        pltpu.sync_copy(acc_sh.at[pl.ds(NS * RPS, TAIL)],
                        out_hbm.at[c, pl.ds(NS * RPS, TAIL)])


def _sc_agg(h, src3, dst3, zeros):
    k = pl.kernel(
        _sc_agg_body,
        out_type=jax.ShapeDtypeStruct((NC, N, D), jnp.float32),
        mesh=_mesh,
        scratch_types=[
            pltpu.VMEM((CPB, CH), jnp.int32),
            pltpu.VMEM((CPB, CH), jnp.int32),
            pltpu.VMEM((CH, D), jnp.float32),
            pltpu.VMEM((CH, D), jnp.float32),
            pltpu.VMEM_SHARED((N + PAD, D), jnp.float32),
            pltpu.SemaphoreType.DMA,
            pltpu.SemaphoreType.DMA,
        ],
        name="sc_gin_agg",
    )
    return k(h, src3, dst3, zeros)


def _bn_relu(z, g, bt):
    m = jnp.mean(z, axis=0, keepdims=True)
    zc = z - m
    v = jnp.mean(zc * zc, axis=0, keepdims=True)
    y = g * zc * lax.rsqrt(v + 1e-5) + bt
    return jnp.maximum(y, 0.0)


def _tc_layer_body(residual, h_ref, p_ref, w_ref, b_ref, g_ref, bt_ref, o_ref):
    h = h_ref[...]
    z = h + p_ref[0] + p_ref[1]
    z = jnp.dot(z, w_ref[...], preferred_element_type=jnp.float32,
                precision=lax.Precision.HIGHEST) + b_ref[...]
    y = _bn_relu(z, g_ref[...], bt_ref[...])
    o_ref[...] = y + h if residual else y


def _tc_layer(h, p, w, b, g, bt, residual):
    return pl.pallas_call(
        functools.partial(_tc_layer_body, residual),
        out_shape=jax.ShapeDtypeStruct((N, D), jnp.float32),
    )(h, p, w, b.reshape(1, D), g.reshape(1, D), bt.reshape(1, D))


def _tc_final_body(h_ref, p_ref, w_ref, b_ref, g_ref, bt_ref, batch_ref,
                   fc1w_ref, fc1b_ref, fc2w_ref, fc2b_ref, o_ref):
    h = h_ref[...]
    z = h + p_ref[0] + p_ref[1]
    z = jnp.dot(z, w_ref[...], preferred_element_type=jnp.float32,
                precision=lax.Precision.HIGHEST) + b_ref[...]
    x3 = _bn_relu(z, g_ref[...], bt_ref[...]) + h
    seg = lax.broadcasted_iota(jnp.int32, (1, G), 1)
    mask = (batch_ref[...] == seg).astype(jnp.float32)
    cnt = jnp.sum(mask, axis=0, keepdims=True)
    mask_n = mask / jnp.maximum(cnt, 1.0)
    pooled = lax.dot_general(mask_n, x3, (((0,), (0,)), ((), ())),
                             preferred_element_type=jnp.float32,
                             precision=lax.Precision.HIGHEST)
    hfc = jnp.maximum(
        jnp.dot(pooled, fc1w_ref[...], preferred_element_type=jnp.float32,
                precision=lax.Precision.HIGHEST) + fc1b_ref[...], 0.0)
    o_ref[...] = jnp.dot(hfc, fc2w_ref[...],
                         preferred_element_type=jnp.float32,
                         precision=lax.Precision.HIGHEST) + fc2b_ref[...]


def _tc_final(h, p, w, b, g, bt, batch, fc1w, fc1b, fc2w, fc2b):
    return pl.pallas_call(
        _tc_final_body,
        out_shape=jax.ShapeDtypeStruct((G, 1), jnp.float32),
    )(h, p, w, b.reshape(1, D), g.reshape(1, D), bt.reshape(1, D),
      batch.reshape(N, 1), fc1w, fc1b.reshape(1, D), fc2w, fc2b.reshape(1, 1))


def kernel(x, edge_index, batch, W1, b1, W2, b2, W3, b3,
           g1, bt1, g2, bt2, g3, bt3, fc1W, fc1b, fc2W, fc2b):
    pad_src = jnp.broadcast_to(jnp.arange(EPAD, dtype=jnp.int32), (NW, EPAD))
    pad_dst = jnp.broadcast_to(N + jnp.arange(EPAD, dtype=jnp.int32) % PAD,
                               (NW, EPAD))
    src3 = jnp.concatenate(
        [edge_index[0].reshape(NW, E // NW), pad_src], axis=1
    ).reshape(NW, NBLK, CPB, CH)
    dst3 = jnp.concatenate(
        [edge_index[1].reshape(NW, E // NW), pad_dst], axis=1
    ).reshape(NW, NBLK, CPB, CH)
    zeros = jnp.zeros((N, D), jnp.float32)

    p1 = _sc_agg(x, src3, dst3, zeros)
    x1 = _tc_layer(x, p1, W1, b1, g1, bt1, residual=False)
    p2 = _sc_agg(x1, src3, dst3, zeros)
    x2 = _tc_layer(x1, p2, W2, b2, g2, bt2, residual=True)
    p3 = _sc_agg(x2, src3, dst3, zeros)
    return _tc_final(x2, p3, W3, b3, g3, bt3, batch,
                     fc1W, fc1b, fc2W, fc2b)


# speedup vs baseline: 10.4924x; 1.0005x over previous
import functools

import jax
import jax.numpy as jnp
from jax import lax
from jax.experimental import pallas as pl
from jax.experimental.pallas import tpu as pltpu
from jax.experimental.pallas import tpu_sc as plsc

N = 10000
E = 320000
D = 128
G = 16
NC = 2
NS = 16
NW = NC * NS
CH = 128
NBUF = 2
PAD = 64
EPW = 10240
EPAD = EPW - E // NW
NBLK = 8
CPB = 10
RPS = 624
TAIL = N - NS * RPS

_mesh = plsc.VectorSubcoreMesh(core_axis_name="c", subcore_axis_name="s")


def _sc_agg_body(h_hbm, src_hbm, dst_hbm, zero_hbm, out_hbm,
                 src_sl, dst_sl, rows, acc_sh, gsems, sem_ia, sem_ib):
    c = lax.axis_index("c")
    s = lax.axis_index("s")
    w = c * NS + s
    pltpu.sync_copy(zero_hbm.at[pl.ds(s * RPS, RPS)],
                    acc_sh.at[pl.ds(s * RPS, RPS)])

    @pl.when(s == NS - 1)
    def _():
        pltpu.sync_copy(zero_hbm.at[pl.ds(NS * RPS, TAIL)],
                        acc_sh.at[pl.ds(NS * RPS, TAIL)])

    def idx_fetch(blk, slab, sem):
        pltpu.async_copy(src_hbm.at[w, blk], src_sl[slab], sem)
        pltpu.async_copy(dst_hbm.at[w, blk], dst_sl[slab], sem)

    def idx_wait(sem):
        pltpu.make_async_copy(src_hbm.at[w, 0], src_sl[0], sem).wait()
        pltpu.make_async_copy(dst_hbm.at[w, 0], dst_sl[0], sem).wait()

    idx_fetch(0, 0, sem_ia)
    plsc.subcore_barrier()

    def gather(slab, j, b):
        pltpu.async_copy(h_hbm.at[src_sl[slab].at[j]], rows[b], gsems[b])

    def gwait(b):
        pltpu.make_async_copy(h_hbm.at[pl.ds(0, CH)], rows[b],
                              gsems[b]).wait()

    def process_block(slab):
        for i in range(NBUF):
            gather(slab, i, i)
        for j in range(CPB):
            b = j % NBUF
            gwait(b)
            pltpu.sync_copy(rows[b], acc_sh.at[dst_sl[slab].at[j]], add=True)
            if j + NBUF < CPB:
                gather(slab, j + NBUF, b)

    @pl.loop(0, NBLK // 2)
    def _(t):
        blk = 2 * t
        idx_wait(sem_ia)
        idx_fetch(blk + 1, 1, sem_ib)
        process_block(0)
        idx_wait(sem_ib)
        idx_fetch(jnp.minimum(blk + 2, NBLK - 1), 0, sem_ia)
        process_block(1)

    idx_wait(sem_ia)
    plsc.subcore_barrier()
    pltpu.sync_copy(acc_sh.at[pl.ds(s * RPS, RPS)],
                    out_hbm.at[c, pl.ds(s * RPS, RPS)])

    @pl.when(s == NS - 1)
    def _():
        pltpu.sync_copy(acc_sh.at[pl.ds(NS * RPS, TAIL)],
                        out_hbm.at[c, pl.ds(NS * RPS, TAIL)])


def _sc_agg(h, src4, dst4, zeros):
    k = pl.kernel(
        _sc_agg_body,
        out_type=jax.ShapeDtypeStruct((NC, N, D), jnp.float32),
        mesh=_mesh,
        scratch_types=[
            [pltpu.VMEM((CPB, CH), jnp.int32) for _ in range(2)],
            [pltpu.VMEM((CPB, CH), jnp.int32) for _ in range(2)],
            [pltpu.VMEM((CH, D), jnp.float32) for _ in range(NBUF)],
            pltpu.VMEM_SHARED((N + PAD, D), jnp.float32),
            [pltpu.SemaphoreType.DMA for _ in range(NBUF)],
            pltpu.SemaphoreType.DMA,
            pltpu.SemaphoreType.DMA,
        ],
        name="sc_gin_agg",
    )
    return k(h, src4, dst4, zeros)


def _bn_relu(z, g, bt):
    m = jnp.mean(z, axis=0, keepdims=True)
    zc = z - m
    v = jnp.mean(zc * zc, axis=0, keepdims=True)
    y = g * zc * lax.rsqrt(v + 1e-5) + bt
    return jnp.maximum(y, 0.0)


def _layer_core(h_ref, p_ref, w_ref, b_ref, g_ref, bt_ref):
    h = h_ref[...]
    z = jnp.dot(h + p_ref[0] + p_ref[1], w_ref[...],
                preferred_element_type=jnp.float32,
                precision=lax.Precision.HIGHEST) + b_ref[...]
    return h, _bn_relu(z, g_ref[...], bt_ref[...])


def _tc_layer_body(residual, h_ref, p_ref, w_ref, b_ref, g_ref, bt_ref,
                   o_ref):
    h, y = _layer_core(h_ref, p_ref, w_ref, b_ref, g_ref, bt_ref)
    o_ref[...] = y + h if residual else y


def _tc_layer(h, p, w, b, g, bt, residual):
    return pl.pallas_call(
        functools.partial(_tc_layer_body, residual),
        out_shape=jax.ShapeDtypeStruct((N, D), jnp.float32),
    )(h, p, w, b.reshape(1, D), g.reshape(1, D), bt.reshape(1, D))


def _tc_final_body(h_ref, p_ref, w_ref, b_ref, g_ref, bt_ref, batch_ref,
                   fc1w_ref, fc1b_ref, fc2w_ref, fc2b_ref, o_ref):
    h, y = _layer_core(h_ref, p_ref, w_ref, b_ref, g_ref, bt_ref)
    x3 = y + h
    seg = lax.broadcasted_iota(jnp.int32, (1, G), 1)
    mask = (batch_ref[...] == seg).astype(jnp.float32)
    cnt = jnp.sum(mask, axis=0, keepdims=True)
    mask_n = mask / jnp.maximum(cnt, 1.0)
    pooled = lax.dot_general(mask_n, x3, (((0,), (0,)), ((), ())),
                             preferred_element_type=jnp.float32,
                             precision=lax.Precision.HIGHEST)
    hfc = jnp.maximum(
        jnp.dot(pooled, fc1w_ref[...], preferred_element_type=jnp.float32,
                precision=lax.Precision.HIGHEST) + fc1b_ref[...], 0.0)
    o_ref[...] = jnp.dot(hfc, fc2w_ref[...],
                         preferred_element_type=jnp.float32,
                         precision=lax.Precision.HIGHEST) + fc2b_ref[...]


def _tc_final(h, p, w, b, g, bt, batch, fc1w, fc1b, fc2w, fc2b):
    return pl.pallas_call(
        _tc_final_body,
        out_shape=jax.ShapeDtypeStruct((G, 1), jnp.float32),
    )(h, p, w, b.reshape(1, D), g.reshape(1, D), bt.reshape(1, D),
      batch.reshape(N, 1), fc1w, fc1b.reshape(1, D), fc2w, fc2b.reshape(1, 1))


def kernel(x, edge_index, batch, W1, b1, W2, b2, W3, b3,
           g1, bt1, g2, bt2, g3, bt3, fc1W, fc1b, fc2W, fc2b):
    pad_src = jnp.broadcast_to(jnp.arange(EPAD, dtype=jnp.int32), (NW, EPAD))
    pad_dst = jnp.broadcast_to(N + jnp.arange(EPAD, dtype=jnp.int32) % PAD,
                               (NW, EPAD))
    src4 = jnp.concatenate(
        [edge_index[0].reshape(NW, E // NW), pad_src], axis=1
    ).reshape(NW, NBLK, CPB, CH)
    dst4 = jnp.concatenate(
        [edge_index[1].reshape(NW, E // NW), pad_dst], axis=1
    ).reshape(NW, NBLK, CPB, CH)
    zeros = jnp.zeros((N, D), jnp.float32)

    p1 = _sc_agg(x, src4, dst4, zeros)
    x1 = _tc_layer(x, p1, W1, b1, g1, bt1, residual=False)
    p2 = _sc_agg(x1, src4, dst4, zeros)
    x2 = _tc_layer(x1, p2, W2, b2, g2, bt2, residual=True)
    p3 = _sc_agg(x2, src4, dst4, zeros)
    return _tc_final(x2, p3, W3, b3, g3, bt3, batch,
                     fc1W, fc1b, fc2W, fc2b)


# speedup vs baseline: 11.1976x; 1.0672x over previous
import functools

import jax
import jax.numpy as jnp
from jax import lax
from jax.experimental import pallas as pl
from jax.experimental.pallas import tpu as pltpu
from jax.experimental.pallas import tpu_sc as plsc

N = 10000
E = 320000
D = 128
G = 16
NC = 2
NS = 16
NW = NC * NS
CH = 64
NBUF = 4
PAD = 64
EPW = 10240
EPAD = EPW - E // NW
NBLK = 8
CPB = 20
RPS = 624
TAIL = N - NS * RPS

_mesh = plsc.VectorSubcoreMesh(core_axis_name="c", subcore_axis_name="s")


def _sc_agg_body(h_hbm, src_hbm, dst_hbm, zero_hbm, out_hbm,
                 src_sl, dst_sl, rows, acc_sh, gsems, sem_ia, sem_ib):
    c = lax.axis_index("c")
    s = lax.axis_index("s")
    w = c * NS + s
    pltpu.sync_copy(zero_hbm.at[pl.ds(s * RPS, RPS)],
                    acc_sh.at[pl.ds(s * RPS, RPS)])

    @pl.when(s == NS - 1)
    def _():
        pltpu.sync_copy(zero_hbm.at[pl.ds(NS * RPS, TAIL)],
                        acc_sh.at[pl.ds(NS * RPS, TAIL)])

    def idx_fetch(blk, slab, sem):
        pltpu.async_copy(src_hbm.at[w, blk], src_sl[slab], sem)
        pltpu.async_copy(dst_hbm.at[w, blk], dst_sl[slab], sem)

    def idx_wait(sem):
        pltpu.make_async_copy(src_hbm.at[w, 0], src_sl[0], sem).wait()
        pltpu.make_async_copy(dst_hbm.at[w, 0], dst_sl[0], sem).wait()

    idx_fetch(0, 0, sem_ia)
    plsc.subcore_barrier()

    def gather(slab, j, b):
        pltpu.async_copy(h_hbm.at[src_sl[slab].at[j]], rows[b], gsems[b])

    def gwait(b):
        pltpu.make_async_copy(h_hbm.at[pl.ds(0, CH)], rows[b],
                              gsems[b]).wait()

    def process_block(slab):
        for i in range(NBUF):
            gather(slab, i, i)
        for j in range(CPB):
            b = j % NBUF
            gwait(b)
            pltpu.sync_copy(rows[b], acc_sh.at[dst_sl[slab].at[j]], add=True)
            if j + NBUF < CPB:
                gather(slab, j + NBUF, b)

    @pl.loop(0, NBLK // 2)
    def _(t):
        blk = 2 * t
        idx_wait(sem_ia)
        idx_fetch(blk + 1, 1, sem_ib)
        process_block(0)
        idx_wait(sem_ib)
        idx_fetch(jnp.minimum(blk + 2, NBLK - 1), 0, sem_ia)
        process_block(1)

    idx_wait(sem_ia)
    plsc.subcore_barrier()
    pltpu.sync_copy(acc_sh.at[pl.ds(s * RPS, RPS)],
                    out_hbm.at[c, pl.ds(s * RPS, RPS)])

    @pl.when(s == NS - 1)
    def _():
        pltpu.sync_copy(acc_sh.at[pl.ds(NS * RPS, TAIL)],
                        out_hbm.at[c, pl.ds(NS * RPS, TAIL)])


def _sc_agg(h, src4, dst4, zeros):
    k = pl.kernel(
        _sc_agg_body,
        out_type=jax.ShapeDtypeStruct((NC, N, D), jnp.float32),
        mesh=_mesh,
        scratch_types=[
            [pltpu.VMEM((CPB, CH), jnp.int32) for _ in range(2)],
            [pltpu.VMEM((CPB, CH), jnp.int32) for _ in range(2)],
            [pltpu.VMEM((CH, D), jnp.float32) for _ in range(NBUF)],
            pltpu.VMEM_SHARED((N + PAD, D), jnp.float32),
            [pltpu.SemaphoreType.DMA for _ in range(NBUF)],
            pltpu.SemaphoreType.DMA,
            pltpu.SemaphoreType.DMA,
        ],
        name="sc_gin_agg",
    )
    return k(h, src4, dst4, zeros)


def _bn_relu(z, g, bt):
    m = jnp.mean(z, axis=0, keepdims=True)
    zc = z - m
    v = jnp.mean(zc * zc, axis=0, keepdims=True)
    y = g * zc * lax.rsqrt(v + 1e-5) + bt
    return jnp.maximum(y, 0.0)


def _layer_core(h_ref, p_ref, w_ref, b_ref, g_ref, bt_ref):
    h = h_ref[...]
    z = jnp.dot(h + p_ref[0] + p_ref[1], w_ref[...],
                preferred_element_type=jnp.float32,
                precision=lax.Precision.HIGHEST) + b_ref[...]
    return h, _bn_relu(z, g_ref[...], bt_ref[...])


def _tc_layer_body(residual, h_ref, p_ref, w_ref, b_ref, g_ref, bt_ref,
                   o_ref):
    h, y = _layer_core(h_ref, p_ref, w_ref, b_ref, g_ref, bt_ref)
    o_ref[...] = y + h if residual else y


def _tc_layer(h, p, w, b, g, bt, residual):
    return pl.pallas_call(
        functools.partial(_tc_layer_body, residual),
        out_shape=jax.ShapeDtypeStruct((N, D), jnp.float32),
    )(h, p, w, b.reshape(1, D), g.reshape(1, D), bt.reshape(1, D))


def _tc_final_body(h_ref, p_ref, w_ref, b_ref, g_ref, bt_ref, batch_ref,
                   fc1w_ref, fc1b_ref, fc2w_ref, fc2b_ref, o_ref):
    h, y = _layer_core(h_ref, p_ref, w_ref, b_ref, g_ref, bt_ref)
    x3 = y + h
    seg = lax.broadcasted_iota(jnp.int32, (1, G), 1)
    mask = (batch_ref[...] == seg).astype(jnp.float32)
    cnt = jnp.sum(mask, axis=0, keepdims=True)
    mask_n = mask / jnp.maximum(cnt, 1.0)
    pooled = lax.dot_general(mask_n, x3, (((0,), (0,)), ((), ())),
                             preferred_element_type=jnp.float32,
                             precision=lax.Precision.HIGHEST)
    hfc = jnp.maximum(
        jnp.dot(pooled, fc1w_ref[...], preferred_element_type=jnp.float32,
                precision=lax.Precision.HIGHEST) + fc1b_ref[...], 0.0)
    o_ref[...] = jnp.dot(hfc, fc2w_ref[...],
                         preferred_element_type=jnp.float32,
                         precision=lax.Precision.HIGHEST) + fc2b_ref[...]


def _tc_final(h, p, w, b, g, bt, batch, fc1w, fc1b, fc2w, fc2b):
    return pl.pallas_call(
        _tc_final_body,
        out_shape=jax.ShapeDtypeStruct((G, 1), jnp.float32),
    )(h, p, w, b.reshape(1, D), g.reshape(1, D), bt.reshape(1, D),
      batch.reshape(N, 1), fc1w, fc1b.reshape(1, D), fc2w, fc2b.reshape(1, 1))


def kernel(x, edge_index, batch, W1, b1, W2, b2, W3, b3,
           g1, bt1, g2, bt2, g3, bt3, fc1W, fc1b, fc2W, fc2b):
    pad_src = jnp.broadcast_to(jnp.arange(EPAD, dtype=jnp.int32), (NW, EPAD))
    pad_dst = jnp.broadcast_to(N + jnp.arange(EPAD, dtype=jnp.int32) % PAD,
                               (NW, EPAD))
    src4 = jnp.concatenate(
        [edge_index[0].reshape(NW, E // NW), pad_src], axis=1
    ).reshape(NW, NBLK, CPB, CH)
    dst4 = jnp.concatenate(
        [edge_index[1].reshape(NW, E // NW), pad_dst], axis=1
    ).reshape(NW, NBLK, CPB, CH)
    zeros = jnp.zeros((N, D), jnp.float32)

    p1 = _sc_agg(x, src4, dst4, zeros)
    x1 = _tc_layer(x, p1, W1, b1, g1, bt1, residual=False)
    p2 = _sc_agg(x1, src4, dst4, zeros)
    x2 = _tc_layer(x1, p2, W2, b2, g2, bt2, residual=True)
    p3 = _sc_agg(x2, src4, dst4, zeros)
    return _tc_final(x2, p3, W3, b3, g3, bt3, batch,
                     fc1W, fc1b, fc2W, fc2b)
